# baseline (speedup 1.0000x reference)
import functools

import jax
import jax.numpy as jnp
from jax import lax
from jax.experimental import pallas as pl
from jax.experimental.pallas import tpu as pltpu
from jax.experimental.pallas import tpu_sc as plsc

BATCH = 4096
SEQ = 200
DIM = 32
HIDDEN = 8
VOCAB = 1000000
PAD_SEQ = 208
CHUNK = PAD_SEQ // 2
LANES = 16

NC = 2
NS = 16
NW = NC * NS
RPW = BATCH // NW


def _sc_pool_body(x_hbm, emb_hbm, out_hbm, idx_v, buf0, buf1, pool_v, sem0, sem1):
    wid = lax.axis_index("s") * NC + lax.axis_index("c")
    base = wid * RPW

    pltpu.sync_copy(x_hbm.at[pl.ds(base, RPW)], idx_v)

    bufs = (buf0, buf1)
    sems = (sem0, sem1)

    for h in range(2):
        pltpu.async_copy(emb_hbm.at[idx_v.at[0, h]], bufs[h], sems[h])

    neg_inf = jnp.full((LANES,), -jnp.inf, dtype=jnp.float32)

    def row_body(r, carry):
        acc_lo = neg_inf
        acc_hi = neg_inf
        nxt = lax.rem(r + 1, RPW)
        for h in range(2):
            buf, sem = bufs[h], sems[h]
            pltpu.make_async_copy(emb_hbm.at[idx_v.at[r, h]], buf, sem).wait()

            def red(t, acc):
                lo = jnp.maximum(acc[0], buf[t, pl.ds(0, LANES)])
                hi = jnp.maximum(acc[1], buf[t, pl.ds(LANES, LANES)])
                return (lo, hi)

            acc_lo, acc_hi = lax.fori_loop(
                0, CHUNK, red, (acc_lo, acc_hi), unroll=8)
            pltpu.async_copy(emb_hbm.at[idx_v.at[nxt, h]], buf, sem)
        pool_v[r, pl.ds(0, LANES)] = acc_lo
        pool_v[r, pl.ds(LANES, LANES)] = acc_hi
        return carry

    lax.fori_loop(0, RPW, row_body, 0)

    for h in range(2):
        pltpu.make_async_copy(emb_hbm.at[idx_v.at[0, h]], bufs[h], sems[h]).wait()

    pltpu.sync_copy(pool_v, out_hbm.at[pl.ds(base, RPW)])


_sc_pool = functools.partial(
    pl.kernel,
    out_type=jax.ShapeDtypeStruct((BATCH, DIM), jnp.float32),
    mesh=plsc.VectorSubcoreMesh(core_axis_name="c", subcore_axis_name="s"),
    scratch_types=[
        pltpu.VMEM((RPW, 2, CHUNK), jnp.int32),
        pltpu.VMEM((CHUNK, DIM), jnp.float32),
        pltpu.VMEM((CHUNK, DIM), jnp.float32),
        pltpu.VMEM((RPW, DIM), jnp.float32),
        pltpu.SemaphoreType.DMA,
        pltpu.SemaphoreType.DMA,
    ],
    compiler_params=pltpu.CompilerParams(use_tc_tiling_on_sc=False),
)(_sc_pool_body)


DT_CB = 2048
DT_BLOCKS = (VOCAB + DT_CB - 1) // DT_CB
VOCAB_PAD = DT_BLOCKS * DT_CB


def _detile_body(src_ref, out_ref):
    t = src_ref[...].T
    u = t.reshape(DT_CB // 4, 4, DIM)
    out_ref[...] = jnp.concatenate([u[:, a, :] for a in range(4)], axis=-1)


def _detile(emb):
    embt = emb.T
    return pl.pallas_call(
        _detile_body,
        grid=(DT_BLOCKS,),
        in_specs=[pl.BlockSpec((DIM, DT_CB), lambda i: (0, i))],
        out_specs=pl.BlockSpec((DT_CB // 4, 4 * DIM), lambda i: (i, 0)),
        out_shape=jax.ShapeDtypeStruct((VOCAB_PAD * DIM // 128, 128), jnp.float32),
    )(embt)


def _mlp_body(pooled_ref, w1t_ref, b1_ref, w2t_ref, b2_ref, out_ref):
    p = pooled_ref[...]
    h = jnp.dot(p, w1t_ref[...], preferred_element_type=jnp.float32)
    h = jnp.maximum(h + b1_ref[...], 0.0)
    z = jnp.dot(h, w2t_ref[...], preferred_element_type=jnp.float32)
    z = z + b2_ref[...]
    out_ref[...] = 1.0 / (1.0 + jnp.exp(-z))


def kernel(x, emb, W1, b1, W2, b2):
    x = x.astype(jnp.int32)
    x_pad = jnp.concatenate([x, x[:, :PAD_SEQ - SEQ]], axis=1)
    x_pad = x_pad.reshape(BATCH, 2, CHUNK)

    table = _detile(emb).reshape(VOCAB_PAD, DIM)
    pooled = _sc_pool(x_pad, table)

    out = pl.pallas_call(
        _mlp_body,
        out_shape=jax.ShapeDtypeStruct((BATCH, 1), jnp.float32),
    )(pooled, W1.T, b1.reshape(1, HIDDEN), W2.T, b2.reshape(1, 1))
    return out


# speedup vs baseline: 1.8985x; 1.8985x over previous
import functools

import jax
import jax.numpy as jnp
from jax import lax
from jax.experimental import pallas as pl
from jax.experimental.pallas import tpu as pltpu
from jax.experimental.pallas import tpu_sc as plsc

BATCH = 4096
SEQ = 200
DIM = 32
HIDDEN = 8
VOCAB = 1000000
PAD_SEQ = 208
CHUNK = PAD_SEQ // 2
LANES = 16

NC = 2
NS = 16
NW = NC * NS
RPW = BATCH // NW
RIF = 4

DT_CB = 8192
DT_Q = DT_CB // 4
DT_BLOCKS = (VOCAB + DT_CB - 1) // DT_CB
VOCAB_PAD = DT_BLOCKS * DT_CB


def _sc_pool_body(x_hbm, emb_hbm, out_hbm, idx_v, bufs_v, pool_v, *sems):
    wid = lax.axis_index("s") * NC + lax.axis_index("c")
    base = wid * RPW

    pltpu.sync_copy(x_hbm.at[pl.ds(base, RPW)], idx_v)

    for r0 in range(RIF):
        for h in range(2):
            b = r0 * 2 + h
            pltpu.async_copy(emb_hbm.at[idx_v.at[r0, h]], bufs_v.at[b], sems[b])

    neg_inf = jnp.full((LANES,), -jnp.inf, dtype=jnp.float32)

    def group_body(g, carry):
        for r_off in range(RIF):
            r = g * RIF + r_off
            nxt = lax.rem(r + RIF, RPW)
            acc_lo = neg_inf
            acc_hi = neg_inf
            for h in range(2):
                b = r_off * 2 + h
                buf = bufs_v.at[b]
                pltpu.make_async_copy(
                    emb_hbm.at[idx_v.at[r, h]], buf, sems[b]).wait()

                def red(t, acc, buf=buf):
                    lo = jnp.maximum(acc[0], buf[t, pl.ds(0, LANES)])
                    hi = jnp.maximum(acc[1], buf[t, pl.ds(LANES, LANES)])
                    return (lo, hi)

                acc_lo, acc_hi = lax.fori_loop(
                    0, CHUNK, red, (acc_lo, acc_hi), unroll=8)
                pltpu.async_copy(emb_hbm.at[idx_v.at[nxt, h]], buf, sems[b])
            pool_v[r, pl.ds(0, LANES)] = acc_lo
            pool_v[r, pl.ds(LANES, LANES)] = acc_hi
        return carry

    lax.fori_loop(0, RPW // RIF, group_body, 0)

    for r0 in range(RIF):
        for h in range(2):
            b = r0 * 2 + h
            pltpu.make_async_copy(
                emb_hbm.at[idx_v.at[r0, h]], bufs_v.at[b], sems[b]).wait()

    pltpu.sync_copy(pool_v, out_hbm.at[pl.ds(base, RPW)])


_sc_pool = functools.partial(
    pl.kernel,
    out_type=jax.ShapeDtypeStruct((BATCH, DIM), jnp.float32),
    mesh=plsc.VectorSubcoreMesh(core_axis_name="c", subcore_axis_name="s"),
    scratch_types=[
        pltpu.VMEM((RPW, 2, CHUNK), jnp.int32),
        pltpu.VMEM((2 * RIF, CHUNK, DIM), jnp.float32),
        pltpu.VMEM((RPW, DIM), jnp.float32),
    ] + [pltpu.SemaphoreType.DMA] * (2 * RIF),
    compiler_params=pltpu.CompilerParams(use_tc_tiling_on_sc=False),
)(_sc_pool_body)


def _detile_body(src_ref, out_ref):
    t = src_ref[...].T
    for a in range(4):
        out_ref[:, DIM * a:DIM * (a + 1)] = t[DT_Q * a:DT_Q * (a + 1), :]


def _detile(emb):
    embt = emb.T
    return pl.pallas_call(
        _detile_body,
        grid=(DT_BLOCKS,),
        in_specs=[pl.BlockSpec((DIM, DT_CB), lambda i: (0, i))],
        out_specs=pl.BlockSpec((DT_CB // 4, 4 * DIM), lambda i: (i, 0)),
        out_shape=jax.ShapeDtypeStruct((VOCAB_PAD * DIM // 128, 128), jnp.float32),
    )(embt)


def _permute_idx(x):
    hi = x & ~(DT_CB - 1)
    return hi + 4 * (x & (DT_Q - 1)) + ((x & (DT_CB - 1)) >> (DT_Q.bit_length() - 1))


def _mlp_body(pooled_ref, w1t_ref, b1_ref, w2t_ref, b2_ref, out_ref):
    p = pooled_ref[...]
    h = jnp.dot(p, w1t_ref[...], preferred_element_type=jnp.float32)
    h = jnp.maximum(h + b1_ref[...], 0.0)
    z = jnp.dot(h, w2t_ref[...], preferred_element_type=jnp.float32)
    z = z + b2_ref[...]
    out_ref[...] = 1.0 / (1.0 + jnp.exp(-z))


def kernel(x, emb, W1, b1, W2, b2):
    x = _permute_idx(x.astype(jnp.int32))
    x_pad = jnp.concatenate([x, x[:, :PAD_SEQ - SEQ]], axis=1)
    x_pad = x_pad.reshape(BATCH, 2, CHUNK)

    table = _detile(emb).reshape(VOCAB_PAD, DIM)
    pooled = _sc_pool(x_pad, table)

    out = pl.pallas_call(
        _mlp_body,
        out_shape=jax.ShapeDtypeStruct((BATCH, 1), jnp.float32),
    )(pooled, W1.T, b1.reshape(1, HIDDEN), W2.T, b2.reshape(1, 1))
    return out
